# baseline (speedup 1.0000x reference)
import functools

import jax
import jax.numpy as jnp
from jax import lax
from jax.experimental import pallas as pl
from jax.experimental.pallas import tpu as pltpu
from jax.experimental.pallas import tpu_sc as plsc

N = 10000
E = 320000
D = 128
H = 64

NC = 2
NS = 16
NW = NC * NS

NPAD = 10240
SLICE = NPAD // NS
CB = 128
CHUNKS = 80
EPW = CHUNKS * CB
EPAD = NW * EPW
NB = 8

R = 1024


def _mesh():
    return plsc.VectorSubcoreMesh(core_axis_name="c", subcore_axis_name="s")


def _sc_deg(dst3, ones_h, zeros_h):
    @functools.partial(
        pl.kernel,
        out_type=jax.ShapeDtypeStruct((NC, NPAD), jnp.float32),
        mesh=_mesh(),
        compiler_params=pltpu.CompilerParams(use_tc_tiling_on_sc=False),
        scratch_types=[
            pltpu.VMEM((CHUNKS + 1, CB), jnp.int32),
            pltpu.VMEM((CB,), jnp.float32),
            pltpu.VMEM((SLICE,), jnp.float32),
            pltpu.VMEM_SHARED((NPAD,), jnp.float32),
            pltpu.SemaphoreType.DMA,
        ],
    )
    def k(dst_h, ones_hbm, zeros_hbm, out_h, dst_v, ones_v, zbuf, acc_sh,
          ssem):
        cid = lax.axis_index("c")
        sid = lax.axis_index("s")
        wid = cid * NS + sid
        pltpu.sync_copy(dst_h.at[wid], dst_v)
        pltpu.sync_copy(ones_hbm, ones_v)
        pltpu.sync_copy(zeros_hbm, zbuf)
        pltpu.sync_copy(zbuf, acc_sh.at[pl.ds(sid * SLICE, SLICE)])
        plsc.subcore_barrier()

        zrows = zbuf.at[pl.ds(0, CB)]
        for _ in range(8):
            pltpu.async_copy(zrows, acc_sh.at[dst_v.at[0]], ssem, add=True)

        def body(j, carry):
            pltpu.async_copy(ones_v, acc_sh.at[dst_v.at[j]], ssem, add=True)
            pltpu.make_async_copy(
                zrows, acc_sh.at[dst_v.at[0]], ssem).wait()
            return carry

        lax.fori_loop(0, CHUNKS, body, 0)
        for _ in range(8):
            pltpu.make_async_copy(
                zrows, acc_sh.at[dst_v.at[0]], ssem).wait()
        plsc.subcore_barrier()
        pltpu.sync_copy(acc_sh.at[pl.ds(sid * SLICE, SLICE)], zbuf)
        pltpu.sync_copy(zbuf, out_h.at[cid, pl.ds(sid * SLICE, SLICE)])

    return k(dst3, ones_h, zeros_h)


def _agg_schedule(tbl_h, src_v, dst_v, bufs, zrows, acc_sh, gsem, ssem,
                  nbuf, look):
    lag = nbuf - look
    for b in range(look):
        pltpu.async_copy(tbl_h.at[src_v.at[b]], bufs.at[b], gsem)
    for _ in range(lag):
        pltpu.async_copy(zrows, acc_sh.at[dst_v.at[CHUNKS]], ssem, add=True)

    def body(g, carry):
        for b in range(nbuf):
            j = g * nbuf + b
            pltpu.make_async_copy(
                tbl_h.at[src_v.at[j]], bufs.at[b], gsem).wait()
            pltpu.async_copy(bufs.at[b], acc_sh.at[dst_v.at[j]], ssem,
                             add=True)
            bb = (b + look) % nbuf
            pltpu.make_async_copy(
                zrows, acc_sh.at[dst_v.at[CHUNKS]], ssem).wait()
            pltpu.async_copy(tbl_h.at[src_v.at[j + look]], bufs.at[bb], gsem)
        return carry

    lax.fori_loop(0, CHUNKS // nbuf, body, 0)
    for b in range(look):
        pltpu.make_async_copy(
            tbl_h.at[src_v.at[CHUNKS]], bufs.at[b], gsem).wait()
    for _ in range(lag):
        pltpu.make_async_copy(
            zrows, acc_sh.at[dst_v.at[CHUNKS]], ssem).wait()


def _sc_agg_rows(y_pad, src3, dst3, zeros_h):
    @functools.partial(
        pl.kernel,
        out_type=jax.ShapeDtypeStruct((NC, NPAD, H), jnp.float32),
        mesh=_mesh(),
        compiler_params=pltpu.CompilerParams(use_tc_tiling_on_sc=False),
        scratch_types=[
            pltpu.VMEM((CHUNKS + 8, CB), jnp.int32),
            pltpu.VMEM((CHUNKS + 1, CB), jnp.int32),
            pltpu.VMEM((5, CB, H), jnp.float32),
            pltpu.VMEM((SLICE // 2, H), jnp.float32),
            pltpu.VMEM_SHARED((NPAD, H), jnp.float32),
            pltpu.SemaphoreType.DMA,
            pltpu.SemaphoreType.DMA,
        ],
    )
    def k(y_h, src_h, dst_h, zeros_hbm, out_h,
          src_v, dst_v, rows_v, zbuf, acc_sh, gsem, ssem):
        cid = lax.axis_index("c")
        sid = lax.axis_index("s")
        wid = cid * NS + sid
        pltpu.sync_copy(src_h.at[wid], src_v)
        pltpu.sync_copy(dst_h.at[wid], dst_v)
        pltpu.sync_copy(zeros_hbm, zbuf)
        half = SLICE // 2
        pltpu.sync_copy(zbuf, acc_sh.at[pl.ds(sid * SLICE, half)])
        pltpu.sync_copy(zbuf, acc_sh.at[pl.ds(sid * SLICE + half, half)])
        plsc.subcore_barrier()

        _agg_schedule(y_h, src_v, dst_v, rows_v, zbuf.at[pl.ds(0, CB)],
                      acc_sh, gsem, ssem, nbuf=5, look=4)

        plsc.subcore_barrier()
        pltpu.sync_copy(acc_sh.at[pl.ds(sid * SLICE, half)], zbuf)
        pltpu.sync_copy(zbuf, out_h.at[cid, pl.ds(sid * SLICE, half)])
        pltpu.sync_copy(acc_sh.at[pl.ds(sid * SLICE + half, half)], zbuf)
        pltpu.sync_copy(zbuf, out_h.at[cid, pl.ds(sid * SLICE + half, half)])

    return k(y_pad, src3, dst3, zeros_h)


def _sc_agg_scalar(z_pad, src3, dst3, zeros_h):
    @functools.partial(
        pl.kernel,
        out_type=jax.ShapeDtypeStruct((NC, NPAD), jnp.float32),
        mesh=_mesh(),
        compiler_params=pltpu.CompilerParams(use_tc_tiling_on_sc=False),
        scratch_types=[
            pltpu.VMEM((CHUNKS + 8, CB), jnp.int32),
            pltpu.VMEM((CHUNKS + 1, CB), jnp.int32),
            pltpu.VMEM((16, CB), jnp.float32),
            pltpu.VMEM((SLICE,), jnp.float32),
            pltpu.VMEM_SHARED((NPAD,), jnp.float32),
            pltpu.SemaphoreType.DMA,
            pltpu.SemaphoreType.DMA,
        ],
    )
    def k(z_h, src_h, dst_h, zeros_hbm, out_h,
          src_v, dst_v, vals_v, zbuf, acc_sh, gsem, ssem):
        cid = lax.axis_index("c")
        sid = lax.axis_index("s")
        wid = cid * NS + sid
        pltpu.sync_copy(src_h.at[wid], src_v)
        pltpu.sync_copy(dst_h.at[wid], dst_v)
        pltpu.sync_copy(zeros_hbm, zbuf)
        pltpu.sync_copy(zbuf, acc_sh.at[pl.ds(sid * SLICE, SLICE)])
        plsc.subcore_barrier()

        _agg_schedule(z_h, src_v, dst_v, vals_v, zbuf.at[pl.ds(0, CB)],
                      acc_sh, gsem, ssem, nbuf=16, look=8)

        plsc.subcore_barrier()
        pltpu.sync_copy(acc_sh.at[pl.ds(sid * SLICE, SLICE)], zbuf)
        pltpu.sync_copy(zbuf, out_h.at[cid, pl.ds(sid * SLICE, SLICE)])

    return k(z_pad, src3, dst3, zeros_h)


def _tc_a_body(x_ref, w_ref, p_ref, y_ref, dis_ref):
    deg = p_ref[:, 0:1] + p_ref[:, 1:2] + 1.0
    dis = lax.rsqrt(deg)
    xw = jnp.dot(x_ref[...], w_ref[...], preferred_element_type=jnp.float32)
    y_ref[...] = xw * dis
    dis_ref[...] = dis


def _tc_a(x_pad, W1, pT):
    return pl.pallas_call(
        _tc_a_body,
        grid=(NPAD // R,),
        in_specs=[
            pl.BlockSpec((R, D), lambda i: (i, 0)),
            pl.BlockSpec((D, H), lambda i: (0, 0)),
            pl.BlockSpec((R, NC), lambda i: (i, 0)),
        ],
        out_specs=[
            pl.BlockSpec((R, H), lambda i: (i, 0)),
            pl.BlockSpec((R, 1), lambda i: (i, 0)),
        ],
        out_shape=[
            jax.ShapeDtypeStruct((NPAD, H), jnp.float32),
            jax.ShapeDtypeStruct((NPAD, 1), jnp.float32),
        ],
    )(x_pad, W1, pT)


def _tc_b_body(a_ref, y_ref, dis_ref, w2_ref, b1_ref, z_ref):
    s = a_ref[0] + a_ref[1] + y_ref[...]
    dis = dis_ref[...]
    h = jnp.maximum(dis * s + b1_ref[...], 0.0)
    z_ref[...] = dis * jnp.dot(h, w2_ref[...],
                               preferred_element_type=jnp.float32)


def _tc_b(acc1, y, dis, W2, b1r):
    return pl.pallas_call(
        _tc_b_body,
        grid=(NPAD // R,),
        in_specs=[
            pl.BlockSpec((NC, R, H), lambda i: (0, i, 0)),
            pl.BlockSpec((R, H), lambda i: (i, 0)),
            pl.BlockSpec((R, 1), lambda i: (i, 0)),
            pl.BlockSpec((H, 1), lambda i: (0, 0)),
            pl.BlockSpec((1, H), lambda i: (0, 0)),
        ],
        out_specs=pl.BlockSpec((R, 1), lambda i: (i, 0)),
        out_shape=jax.ShapeDtypeStruct((NPAD, 1), jnp.float32),
    )(acc1, y, dis, W2, b1r)


def _tc_c_body(aT_ref, z_ref, dis_ref, b2_ref, out_ref):
    s = aT_ref[:, 0:1] + aT_ref[:, 1:2] + z_ref[...]
    out_ref[...] = dis_ref[...] * s + b2_ref[...]


def _tc_c(a2T, z, dis, b2r):
    return pl.pallas_call(
        _tc_c_body,
        grid=(NPAD // R,),
        in_specs=[
            pl.BlockSpec((R, NC), lambda i: (i, 0)),
            pl.BlockSpec((R, 1), lambda i: (i, 0)),
            pl.BlockSpec((R, 1), lambda i: (i, 0)),
            pl.BlockSpec((1, 1), lambda i: (0, 0)),
        ],
        out_specs=pl.BlockSpec((R, 1), lambda i: (i, 0)),
        out_shape=jax.ShapeDtypeStruct((NPAD, 1), jnp.float32),
    )(a2T, z, dis, b2r)


def kernel(x, edge_index, W1, b1, W2, b2):
    src = edge_index[0].astype(jnp.int32)
    dst = edge_index[1].astype(jnp.int32)
    npd = NPAD - N
    pad = EPAD - E
    pad_idx = jnp.arange(pad, dtype=jnp.int32)
    src3 = jnp.concatenate([src, pad_idx % N]).reshape(NW, CHUNKS, CB)
    src3 = jnp.concatenate([src3, src3[:, :8, :]], axis=1)
    dst3 = jnp.concatenate([dst, N + pad_idx % npd]).reshape(NW, CHUNKS, CB)
    dummy_dst = jnp.broadcast_to(N + jnp.arange(CB, dtype=jnp.int32) % npd,
                                 (NW, 1, CB))
    dst3 = jnp.concatenate([dst3, dummy_dst], axis=1)

    ones_h = jnp.ones((CB,), jnp.float32)
    zeros1 = jnp.zeros((SLICE,), jnp.float32)
    zeros2 = jnp.zeros((SLICE // 2, H), jnp.float32)

    degp = _sc_deg(dst3, ones_h, zeros1)
    x_pad = jnp.pad(x, ((0, npd), (0, 0)))
    y, dis = _tc_a(x_pad, W1, degp.T)
    acc1 = _sc_agg_rows(y, src3, dst3, zeros2)
    z = _tc_b(acc1, y, dis, W2, b1.reshape(1, H))
    acc2 = _sc_agg_scalar(z.reshape(NPAD), src3, dst3, zeros1)
    out = _tc_c(acc2.T, z, dis, b2.reshape(1, 1))
    return out[:N]


# speedup vs baseline: 1.0303x; 1.0303x over previous
import functools

import jax
import jax.numpy as jnp
from jax import lax
from jax.experimental import pallas as pl
from jax.experimental.pallas import tpu as pltpu
from jax.experimental.pallas import tpu_sc as plsc

N = 10000
E = 320000
D = 128
H = 64

NC = 2
NS = 16
NW = NC * NS

NPAD = 10240
SLICE = NPAD // NS
CB = 128
CHUNKS = 80
EPW = CHUNKS * CB
EPAD = NW * EPW
NB = 8

R = 2048


def _mesh():
    return plsc.VectorSubcoreMesh(core_axis_name="c", subcore_axis_name="s")


def _sc_deg(dst3, ones_h, zeros_h):
    @functools.partial(
        pl.kernel,
        out_type=jax.ShapeDtypeStruct((NC, NPAD), jnp.float32),
        mesh=_mesh(),
        compiler_params=pltpu.CompilerParams(use_tc_tiling_on_sc=False),
        scratch_types=[
            pltpu.VMEM((CHUNKS + 1, CB), jnp.int32),
            pltpu.VMEM((CB,), jnp.float32),
            pltpu.VMEM((SLICE,), jnp.float32),
            pltpu.VMEM_SHARED((NPAD,), jnp.float32),
            pltpu.SemaphoreType.DMA,
        ],
    )
    def k(dst_h, ones_hbm, zeros_hbm, out_h, dst_v, ones_v, zbuf, acc_sh,
          ssem):
        cid = lax.axis_index("c")
        sid = lax.axis_index("s")
        wid = cid * NS + sid
        pltpu.sync_copy(dst_h.at[wid], dst_v)
        pltpu.sync_copy(ones_hbm, ones_v)
        pltpu.sync_copy(zeros_hbm, zbuf)
        pltpu.sync_copy(zbuf, acc_sh.at[pl.ds(sid * SLICE, SLICE)])
        plsc.subcore_barrier()

        zrows = zbuf.at[pl.ds(0, CB)]
        for _ in range(8):
            pltpu.async_copy(zrows, acc_sh.at[dst_v.at[0]], ssem, add=True)

        def body(j, carry):
            pltpu.async_copy(ones_v, acc_sh.at[dst_v.at[j]], ssem, add=True)
            pltpu.make_async_copy(
                zrows, acc_sh.at[dst_v.at[0]], ssem).wait()
            return carry

        lax.fori_loop(0, CHUNKS, body, 0)
        for _ in range(8):
            pltpu.make_async_copy(
                zrows, acc_sh.at[dst_v.at[0]], ssem).wait()
        plsc.subcore_barrier()
        pltpu.sync_copy(acc_sh.at[pl.ds(sid * SLICE, SLICE)], zbuf)
        pltpu.sync_copy(zbuf, out_h.at[cid, pl.ds(sid * SLICE, SLICE)])

    return k(dst3, ones_h, zeros_h)


def _agg_schedule(tbl_h, src_v, dst_v, bufs, zrows, acc_sh, gsem, ssem,
                  nbuf, look):
    lag = nbuf - look
    for b in range(look):
        pltpu.async_copy(tbl_h.at[src_v.at[b]], bufs.at[b], gsem)
    for _ in range(lag):
        pltpu.async_copy(zrows, acc_sh.at[dst_v.at[CHUNKS]], ssem, add=True)

    def body(g, carry):
        for b in range(nbuf):
            j = g * nbuf + b
            pltpu.make_async_copy(
                tbl_h.at[src_v.at[j]], bufs.at[b], gsem).wait()
            pltpu.async_copy(bufs.at[b], acc_sh.at[dst_v.at[j]], ssem,
                             add=True)
            bb = (b + look) % nbuf
            pltpu.make_async_copy(
                zrows, acc_sh.at[dst_v.at[CHUNKS]], ssem).wait()
            pltpu.async_copy(tbl_h.at[src_v.at[j + look]], bufs.at[bb], gsem)
        return carry

    lax.fori_loop(0, CHUNKS // nbuf, body, 0)
    for b in range(look):
        pltpu.make_async_copy(
            tbl_h.at[src_v.at[CHUNKS]], bufs.at[b], gsem).wait()
    for _ in range(lag):
        pltpu.make_async_copy(
            zrows, acc_sh.at[dst_v.at[CHUNKS]], ssem).wait()


def _sc_agg_rows(y_pad, src3, dst3, zeros_h):
    @functools.partial(
        pl.kernel,
        out_type=jax.ShapeDtypeStruct((NC, NPAD, H), jnp.float32),
        mesh=_mesh(),
        compiler_params=pltpu.CompilerParams(use_tc_tiling_on_sc=False),
        scratch_types=[
            pltpu.VMEM((CHUNKS + 8, CB), jnp.int32),
            pltpu.VMEM((CHUNKS + 1, CB), jnp.int32),
            pltpu.VMEM((5, CB, H), jnp.float32),
            pltpu.VMEM((SLICE // 2, H), jnp.float32),
            pltpu.VMEM_SHARED((NPAD, H), jnp.float32),
            pltpu.SemaphoreType.DMA,
            pltpu.SemaphoreType.DMA,
        ],
    )
    def k(y_h, src_h, dst_h, zeros_hbm, out_h,
          src_v, dst_v, rows_v, zbuf, acc_sh, gsem, ssem):
        cid = lax.axis_index("c")
        sid = lax.axis_index("s")
        wid = cid * NS + sid
        pltpu.sync_copy(src_h.at[wid], src_v)
        pltpu.sync_copy(dst_h.at[wid], dst_v)
        pltpu.sync_copy(zeros_hbm, zbuf)
        half = SLICE // 2
        pltpu.sync_copy(zbuf, acc_sh.at[pl.ds(sid * SLICE, half)])
        pltpu.sync_copy(zbuf, acc_sh.at[pl.ds(sid * SLICE + half, half)])
        plsc.subcore_barrier()

        _agg_schedule(y_h, src_v, dst_v, rows_v, zbuf.at[pl.ds(0, CB)],
                      acc_sh, gsem, ssem, nbuf=5, look=3)

        plsc.subcore_barrier()
        pltpu.sync_copy(acc_sh.at[pl.ds(sid * SLICE, half)], zbuf)
        pltpu.sync_copy(zbuf, out_h.at[cid, pl.ds(sid * SLICE, half)])
        pltpu.sync_copy(acc_sh.at[pl.ds(sid * SLICE + half, half)], zbuf)
        pltpu.sync_copy(zbuf, out_h.at[cid, pl.ds(sid * SLICE + half, half)])

    return k(y_pad, src3, dst3, zeros_h)


def _sc_agg_scalar(z_pad, src3, dst3, zeros_h):
    @functools.partial(
        pl.kernel,
        out_type=jax.ShapeDtypeStruct((NC, NPAD), jnp.float32),
        mesh=_mesh(),
        compiler_params=pltpu.CompilerParams(use_tc_tiling_on_sc=False),
        scratch_types=[
            pltpu.VMEM((CHUNKS + 8, CB), jnp.int32),
            pltpu.VMEM((CHUNKS + 1, CB), jnp.int32),
            pltpu.VMEM((16, CB), jnp.float32),
            pltpu.VMEM((SLICE,), jnp.float32),
            pltpu.VMEM_SHARED((NPAD,), jnp.float32),
            pltpu.SemaphoreType.DMA,
            pltpu.SemaphoreType.DMA,
        ],
    )
    def k(z_h, src_h, dst_h, zeros_hbm, out_h,
          src_v, dst_v, vals_v, zbuf, acc_sh, gsem, ssem):
        cid = lax.axis_index("c")
        sid = lax.axis_index("s")
        wid = cid * NS + sid
        pltpu.sync_copy(src_h.at[wid], src_v)
        pltpu.sync_copy(dst_h.at[wid], dst_v)
        pltpu.sync_copy(zeros_hbm, zbuf)
        pltpu.sync_copy(zbuf, acc_sh.at[pl.ds(sid * SLICE, SLICE)])
        plsc.subcore_barrier()

        _agg_schedule(z_h, src_v, dst_v, vals_v, zbuf.at[pl.ds(0, CB)],
                      acc_sh, gsem, ssem, nbuf=16, look=8)

        plsc.subcore_barrier()
        pltpu.sync_copy(acc_sh.at[pl.ds(sid * SLICE, SLICE)], zbuf)
        pltpu.sync_copy(zbuf, out_h.at[cid, pl.ds(sid * SLICE, SLICE)])

    return k(z_pad, src3, dst3, zeros_h)


def _tc_a_body(x_ref, w_ref, p_ref, y_ref, dis_ref):
    deg = p_ref[:, 0:1] + p_ref[:, 1:2] + 1.0
    dis = lax.rsqrt(deg)
    xw = jnp.dot(x_ref[...], w_ref[...], preferred_element_type=jnp.float32)
    y_ref[...] = xw * dis
    dis_ref[...] = dis


def _tc_a(x_pad, W1, pT):
    return pl.pallas_call(
        _tc_a_body,
        grid=(NPAD // R,),
        in_specs=[
            pl.BlockSpec((R, D), lambda i: (i, 0)),
            pl.BlockSpec((D, H), lambda i: (0, 0)),
            pl.BlockSpec((R, NC), lambda i: (i, 0)),
        ],
        out_specs=[
            pl.BlockSpec((R, H), lambda i: (i, 0)),
            pl.BlockSpec((R, 1), lambda i: (i, 0)),
        ],
        out_shape=[
            jax.ShapeDtypeStruct((NPAD, H), jnp.float32),
            jax.ShapeDtypeStruct((NPAD, 1), jnp.float32),
        ],
    )(x_pad, W1, pT)


def _tc_b_body(a_ref, y_ref, dis_ref, w2_ref, b1_ref, z_ref):
    s = a_ref[0] + a_ref[1] + y_ref[...]
    dis = dis_ref[...]
    h = jnp.maximum(dis * s + b1_ref[...], 0.0)
    z_ref[...] = dis * jnp.dot(h, w2_ref[...],
                               preferred_element_type=jnp.float32)


def _tc_b(acc1, y, dis, W2, b1r):
    return pl.pallas_call(
        _tc_b_body,
        grid=(NPAD // R,),
        in_specs=[
            pl.BlockSpec((NC, R, H), lambda i: (0, i, 0)),
            pl.BlockSpec((R, H), lambda i: (i, 0)),
            pl.BlockSpec((R, 1), lambda i: (i, 0)),
            pl.BlockSpec((H, 1), lambda i: (0, 0)),
            pl.BlockSpec((1, H), lambda i: (0, 0)),
        ],
        out_specs=pl.BlockSpec((R, 1), lambda i: (i, 0)),
        out_shape=jax.ShapeDtypeStruct((NPAD, 1), jnp.float32),
    )(acc1, y, dis, W2, b1r)


def _tc_c_body(aT_ref, z_ref, dis_ref, b2_ref, out_ref):
    s = aT_ref[:, 0:1] + aT_ref[:, 1:2] + z_ref[...]
    out_ref[...] = dis_ref[...] * s + b2_ref[...]


def _tc_c(a2T, z, dis, b2r):
    return pl.pallas_call(
        _tc_c_body,
        grid=(NPAD // R,),
        in_specs=[
            pl.BlockSpec((R, NC), lambda i: (i, 0)),
            pl.BlockSpec((R, 1), lambda i: (i, 0)),
            pl.BlockSpec((R, 1), lambda i: (i, 0)),
            pl.BlockSpec((1, 1), lambda i: (0, 0)),
        ],
        out_specs=pl.BlockSpec((R, 1), lambda i: (i, 0)),
        out_shape=jax.ShapeDtypeStruct((NPAD, 1), jnp.float32),
    )(a2T, z, dis, b2r)


def kernel(x, edge_index, W1, b1, W2, b2):
    src = edge_index[0].astype(jnp.int32)
    dst = edge_index[1].astype(jnp.int32)
    npd = NPAD - N
    pad = EPAD - E
    pad_idx = jnp.arange(pad, dtype=jnp.int32)
    src3 = jnp.concatenate([src, pad_idx % N]).reshape(NW, CHUNKS, CB)
    src3 = jnp.concatenate([src3, src3[:, :8, :]], axis=1)
    dst3 = jnp.concatenate([dst, N + pad_idx % npd]).reshape(NW, CHUNKS, CB)
    dummy_dst = jnp.broadcast_to(N + jnp.arange(CB, dtype=jnp.int32) % npd,
                                 (NW, 1, CB))
    dst3 = jnp.concatenate([dst3, dummy_dst], axis=1)

    ones_h = jnp.ones((CB,), jnp.float32)
    zeros1 = jnp.zeros((SLICE,), jnp.float32)
    zeros2 = jnp.zeros((SLICE // 2, H), jnp.float32)

    degp = _sc_deg(dst3, ones_h, zeros1)
    x_pad = jnp.pad(x, ((0, npd), (0, 0)))
    y, dis = _tc_a(x_pad, W1, degp.T)
    acc1 = _sc_agg_rows(y, src3, dst3, zeros2)
    z = _tc_b(acc1, y, dis, W2, b1.reshape(1, H))
    acc2 = _sc_agg_scalar(z.reshape(NPAD), src3, dst3, zeros1)
    out = _tc_c(acc2.T, z, dis, b2.reshape(1, 1))
    return out[:N]


# speedup vs baseline: 1.0500x; 1.0191x over previous
import functools

import jax
import jax.numpy as jnp
from jax import lax
from jax.experimental import pallas as pl
from jax.experimental.pallas import tpu as pltpu
from jax.experimental.pallas import tpu_sc as plsc

N = 10000
E = 320000
D = 128
H = 64

NC = 2
NS = 16
NW = NC * NS

NPAD = 10240
SLICE = NPAD // NS
CB = 128
CHUNKS = 80
EPW = CHUNKS * CB
EPAD = NW * EPW
NB = 8

R = 5120


def _mesh():
    return plsc.VectorSubcoreMesh(core_axis_name="c", subcore_axis_name="s")


def _sc_deg(dst3, ones_h, zeros_h):
    @functools.partial(
        pl.kernel,
        out_type=jax.ShapeDtypeStruct((NC, NPAD), jnp.float32),
        mesh=_mesh(),
        compiler_params=pltpu.CompilerParams(use_tc_tiling_on_sc=False),
        scratch_types=[
            pltpu.VMEM((CHUNKS + 1, CB), jnp.int32),
            pltpu.VMEM((CB,), jnp.float32),
            pltpu.VMEM((SLICE,), jnp.float32),
            pltpu.VMEM_SHARED((NPAD,), jnp.float32),
            pltpu.SemaphoreType.DMA,
        ],
    )
    def k(dst_h, ones_hbm, zeros_hbm, out_h, dst_v, ones_v, zbuf, acc_sh,
          ssem):
        cid = lax.axis_index("c")
        sid = lax.axis_index("s")
        wid = cid * NS + sid
        pltpu.sync_copy(dst_h.at[wid], dst_v)
        pltpu.sync_copy(ones_hbm, ones_v)
        pltpu.sync_copy(zeros_hbm, zbuf)
        pltpu.sync_copy(zbuf, acc_sh.at[pl.ds(sid * SLICE, SLICE)])
        plsc.subcore_barrier()

        zrows = zbuf.at[pl.ds(0, CB)]
        for _ in range(8):
            pltpu.async_copy(zrows, acc_sh.at[dst_v.at[0]], ssem, add=True)

        def body(j, carry):
            pltpu.async_copy(ones_v, acc_sh.at[dst_v.at[j]], ssem, add=True)
            pltpu.make_async_copy(
                zrows, acc_sh.at[dst_v.at[0]], ssem).wait()
            return carry

        lax.fori_loop(0, CHUNKS, body, 0)
        for _ in range(8):
            pltpu.make_async_copy(
                zrows, acc_sh.at[dst_v.at[0]], ssem).wait()
        plsc.subcore_barrier()
        pltpu.sync_copy(acc_sh.at[pl.ds(sid * SLICE, SLICE)], zbuf)
        pltpu.sync_copy(zbuf, out_h.at[cid, pl.ds(sid * SLICE, SLICE)])

    return k(dst3, ones_h, zeros_h)


def _agg_schedule(tbl_h, src_v, dst_v, bufs, zrows, acc_sh, gsem, ssem,
                  nbuf, look):
    lag = nbuf - look
    for b in range(look):
        pltpu.async_copy(tbl_h.at[src_v.at[b]], bufs.at[b], gsem)
    for _ in range(lag):
        pltpu.async_copy(zrows, acc_sh.at[dst_v.at[CHUNKS]], ssem, add=True)

    def body(g, carry):
        for b in range(nbuf):
            j = g * nbuf + b
            pltpu.make_async_copy(
                tbl_h.at[src_v.at[j]], bufs.at[b], gsem).wait()
            pltpu.async_copy(bufs.at[b], acc_sh.at[dst_v.at[j]], ssem,
                             add=True)
            bb = (b + look) % nbuf
            pltpu.make_async_copy(
                zrows, acc_sh.at[dst_v.at[CHUNKS]], ssem).wait()
            pltpu.async_copy(tbl_h.at[src_v.at[j + look]], bufs.at[bb], gsem)
        return carry

    lax.fori_loop(0, CHUNKS // nbuf, body, 0)
    for b in range(look):
        pltpu.make_async_copy(
            tbl_h.at[src_v.at[CHUNKS]], bufs.at[b], gsem).wait()
    for _ in range(lag):
        pltpu.make_async_copy(
            zrows, acc_sh.at[dst_v.at[CHUNKS]], ssem).wait()


def _sc_agg_rows(y_pad, src3, dst3, zeros_h):
    @functools.partial(
        pl.kernel,
        out_type=jax.ShapeDtypeStruct((NC, NPAD, H), jnp.float32),
        mesh=_mesh(),
        compiler_params=pltpu.CompilerParams(use_tc_tiling_on_sc=False),
        scratch_types=[
            pltpu.VMEM((CHUNKS + 8, CB), jnp.int32),
            pltpu.VMEM((CHUNKS + 1, CB), jnp.int32),
            pltpu.VMEM((5, CB, H), jnp.float32),
            pltpu.VMEM((SLICE // 2, H), jnp.float32),
            pltpu.VMEM_SHARED((NPAD, H), jnp.float32),
            pltpu.SemaphoreType.DMA,
            pltpu.SemaphoreType.DMA,
        ],
    )
    def k(y_h, src_h, dst_h, zeros_hbm, out_h,
          src_v, dst_v, rows_v, zbuf, acc_sh, gsem, ssem):
        cid = lax.axis_index("c")
        sid = lax.axis_index("s")
        wid = cid * NS + sid
        pltpu.sync_copy(src_h.at[wid], src_v)
        pltpu.sync_copy(dst_h.at[wid], dst_v)
        pltpu.sync_copy(zeros_hbm, zbuf)
        half = SLICE // 2
        pltpu.sync_copy(zbuf, acc_sh.at[pl.ds(sid * SLICE, half)])
        pltpu.sync_copy(zbuf, acc_sh.at[pl.ds(sid * SLICE + half, half)])
        plsc.subcore_barrier()

        _agg_schedule(y_h, src_v, dst_v, rows_v, zbuf.at[pl.ds(0, CB)],
                      acc_sh, gsem, ssem, nbuf=5, look=3)

        plsc.subcore_barrier()
        pltpu.sync_copy(acc_sh.at[pl.ds(sid * SLICE, half)], zbuf)
        pltpu.sync_copy(zbuf, out_h.at[cid, pl.ds(sid * SLICE, half)])
        pltpu.sync_copy(acc_sh.at[pl.ds(sid * SLICE + half, half)], zbuf)
        pltpu.sync_copy(zbuf, out_h.at[cid, pl.ds(sid * SLICE + half, half)])

    return k(y_pad, src3, dst3, zeros_h)


def _sc_agg_scalar(z_pad, src3, dst3, zeros_h):
    @functools.partial(
        pl.kernel,
        out_type=jax.ShapeDtypeStruct((NC, NPAD), jnp.float32),
        mesh=_mesh(),
        compiler_params=pltpu.CompilerParams(use_tc_tiling_on_sc=False),
        scratch_types=[
            pltpu.VMEM((CHUNKS + 8, CB), jnp.int32),
            pltpu.VMEM((CHUNKS + 1, CB), jnp.int32),
            pltpu.VMEM((16, CB), jnp.float32),
            pltpu.VMEM((SLICE,), jnp.float32),
            pltpu.VMEM_SHARED((NPAD,), jnp.float32),
            pltpu.SemaphoreType.DMA,
            pltpu.SemaphoreType.DMA,
        ],
    )
    def k(z_h, src_h, dst_h, zeros_hbm, out_h,
          src_v, dst_v, vals_v, zbuf, acc_sh, gsem, ssem):
        cid = lax.axis_index("c")
        sid = lax.axis_index("s")
        wid = cid * NS + sid
        pltpu.sync_copy(src_h.at[wid], src_v)
        pltpu.sync_copy(dst_h.at[wid], dst_v)
        pltpu.sync_copy(zeros_hbm, zbuf)
        pltpu.sync_copy(zbuf, acc_sh.at[pl.ds(sid * SLICE, SLICE)])
        plsc.subcore_barrier()

        _agg_schedule(z_h, src_v, dst_v, vals_v, zbuf.at[pl.ds(0, CB)],
                      acc_sh, gsem, ssem, nbuf=16, look=8)

        plsc.subcore_barrier()
        pltpu.sync_copy(acc_sh.at[pl.ds(sid * SLICE, SLICE)], zbuf)
        pltpu.sync_copy(zbuf, out_h.at[cid, pl.ds(sid * SLICE, SLICE)])

    return k(z_pad, src3, dst3, zeros_h)


def _tc_a_body(x_ref, w_ref, p_ref, y_ref, dis_ref):
    deg = p_ref[:, 0:1] + p_ref[:, 1:2] + 1.0
    dis = lax.rsqrt(deg)
    xw = jnp.dot(x_ref[...], w_ref[...], preferred_element_type=jnp.float32)
    y_ref[...] = xw * dis
    dis_ref[...] = dis


def _tc_a(x_pad, W1, pT):
    return pl.pallas_call(
        _tc_a_body,
        grid=(NPAD // R,),
        in_specs=[
            pl.BlockSpec((R, D), lambda i: (i, 0)),
            pl.BlockSpec((D, H), lambda i: (0, 0)),
            pl.BlockSpec((R, NC), lambda i: (i, 0)),
        ],
        out_specs=[
            pl.BlockSpec((R, H), lambda i: (i, 0)),
            pl.BlockSpec((R, 1), lambda i: (i, 0)),
        ],
        out_shape=[
            jax.ShapeDtypeStruct((NPAD, H), jnp.float32),
            jax.ShapeDtypeStruct((NPAD, 1), jnp.float32),
        ],
    )(x_pad, W1, pT)


def _tc_b_body(a_ref, y_ref, dis_ref, w2_ref, b1_ref, z_ref):
    s = a_ref[0] + a_ref[1] + y_ref[...]
    dis = dis_ref[...]
    h = jnp.maximum(dis * s + b1_ref[...], 0.0)
    z_ref[...] = dis * jnp.dot(h, w2_ref[...],
                               preferred_element_type=jnp.float32)


def _tc_b(acc1, y, dis, W2, b1r):
    return pl.pallas_call(
        _tc_b_body,
        grid=(NPAD // R,),
        in_specs=[
            pl.BlockSpec((NC, R, H), lambda i: (0, i, 0)),
            pl.BlockSpec((R, H), lambda i: (i, 0)),
            pl.BlockSpec((R, 1), lambda i: (i, 0)),
            pl.BlockSpec((H, 1), lambda i: (0, 0)),
            pl.BlockSpec((1, H), lambda i: (0, 0)),
        ],
        out_specs=pl.BlockSpec((R, 1), lambda i: (i, 0)),
        out_shape=jax.ShapeDtypeStruct((NPAD, 1), jnp.float32),
    )(acc1, y, dis, W2, b1r)


def _tc_c_body(aT_ref, z_ref, dis_ref, b2_ref, out_ref):
    s = aT_ref[:, 0:1] + aT_ref[:, 1:2] + z_ref[...]
    out_ref[...] = dis_ref[...] * s + b2_ref[...]


def _tc_c(a2T, z, dis, b2r):
    return pl.pallas_call(
        _tc_c_body,
        grid=(NPAD // R,),
        in_specs=[
            pl.BlockSpec((R, NC), lambda i: (i, 0)),
            pl.BlockSpec((R, 1), lambda i: (i, 0)),
            pl.BlockSpec((R, 1), lambda i: (i, 0)),
            pl.BlockSpec((1, 1), lambda i: (0, 0)),
        ],
        out_specs=pl.BlockSpec((R, 1), lambda i: (i, 0)),
        out_shape=jax.ShapeDtypeStruct((NPAD, 1), jnp.float32),
    )(a2T, z, dis, b2r)


def kernel(x, edge_index, W1, b1, W2, b2):
    src = edge_index[0].astype(jnp.int32)
    dst = edge_index[1].astype(jnp.int32)
    npd = NPAD - N
    pad = EPAD - E
    pad_idx = jnp.arange(pad, dtype=jnp.int32)
    src3 = jnp.concatenate([src, pad_idx % N]).reshape(NW, CHUNKS, CB)
    src3 = jnp.concatenate([src3, src3[:, :8, :]], axis=1)
    dst3 = jnp.concatenate([dst, N + pad_idx % npd]).reshape(NW, CHUNKS, CB)
    dummy_dst = jnp.broadcast_to(N + jnp.arange(CB, dtype=jnp.int32) % npd,
                                 (NW, 1, CB))
    dst3 = jnp.concatenate([dst3, dummy_dst], axis=1)

    ones_h = jnp.ones((CB,), jnp.float32)
    zeros1 = jnp.zeros((SLICE,), jnp.float32)
    zeros2 = jnp.zeros((SLICE // 2, H), jnp.float32)

    degp = _sc_deg(dst3, ones_h, zeros1)
    x_pad = jnp.pad(x, ((0, npd), (0, 0)))
    y, dis = _tc_a(x_pad, W1, degp.T)
    acc1 = _sc_agg_rows(y, src3, dst3, zeros2)
    z = _tc_b(acc1, y, dis, W2, b1.reshape(1, H))
    acc2 = _sc_agg_scalar(z.reshape(NPAD), src3, dst3, zeros1)
    out = _tc_c(acc2.T, z, dis, b2.reshape(1, 1))
    return out[:N]
